# baseline (speedup 1.0000x reference)
import jax
import jax.numpy as jnp
import numpy as np
from jax.experimental import pallas as pl

_N_CLASSES = 64
_SEQ_C = 6
_SELECTED_MAPS = [[3, 17, 42], [5, 9, 28, 51], [0, 12, 33], [7, 21, 44, 60], [2, 14, 39], [8, 26, 55, 63]]
_MIS_VAL = 0.5
_TB = 512


def _map_mask_padded():
    m = np.zeros((8, _N_CLASSES), dtype=np.float32)
    for i, vals in enumerate(_SELECTED_MAPS):
        m[1 + i, vals] = 1.0
    return jnp.asarray(m)


def _loss_kernel(logits_ref, primary_ref, aux_ref, mm_ref, out_ref):
    x = logits_ref[...]
    prim = primary_ref[...]
    a2 = aux_ref[:, 0]
    match = aux_ref[:, 1]

    c = jax.lax.broadcasted_iota(jnp.int32, x.shape, 2)
    s = jax.lax.broadcasted_iota(jnp.int32, x.shape, 1)

    eq_p = c == prim[:, :, None]
    t_match = eq_p.astype(jnp.float32) + (
        (c == a2[:, None, None]) & (s == 0)
    ).astype(jnp.float32)
    t_nonmatch = jnp.where(eq_p, _MIS_VAL, mm_ref[...][None, :, :])
    mid = (s >= 1) & (s <= 6)
    nonmatching = (match == 0)[:, None, None]
    t = jnp.where(mid & nonmatching, t_nonmatch, t_match)

    bce = jnp.maximum(x, 0.0) - x * t + jnp.log1p(jnp.exp(-jnp.abs(x)))
    out_ref[...] = jnp.sum(bce, axis=2)


def kernel(logits, b_train_phrase, b_attitude_1, b_attitude_2, b_compare, b_matching):
    B = logits.shape[0]
    primary = jnp.concatenate(
        [b_attitude_1, b_compare, b_train_phrase[:, -1:]], axis=1
    ).astype(jnp.int32)
    aux = jnp.concatenate([b_attitude_2, b_matching], axis=1).astype(jnp.int32)
    mm = _map_mask_padded()

    grid = (B // _TB,)
    return pl.pallas_call(
        _loss_kernel,
        grid=grid,
        in_specs=[
            pl.BlockSpec((_TB, 8, _N_CLASSES), lambda i: (i, 0, 0)),
            pl.BlockSpec((_TB, 8), lambda i: (i, 0)),
            pl.BlockSpec((_TB, 2), lambda i: (i, 0)),
            pl.BlockSpec((8, _N_CLASSES), lambda i: (0, 0)),
        ],
        out_specs=pl.BlockSpec((_TB, 8), lambda i: (i, 0)),
        out_shape=jax.ShapeDtypeStruct((B, 8), jnp.float32),
    )(logits, primary, aux, mm)
